# baseline (speedup 1.0000x reference)
import functools

import jax
import jax.numpy as jnp
from jax import lax
from jax.experimental import pallas as pl
from jax.experimental.pallas import tpu as pltpu
from jax.experimental.pallas import tpu_sc as plsc

B = 16384
GMF_DIM = 64
MLP_DIM = 32
CH = 128
LN = 16
NW = 32
MLP_SPLIT = 12544
MLP_B2 = 1792
MLP_NBLK = 7
GMF_SPLIT = 25088
GMF_B2 = 3584
GMF_NBLK = 7


def _pack_bf16_halves(t, d):
    hw = d // 2
    lo = lax.bitcast_convert_type(t[:, :hw].astype(jnp.bfloat16), jnp.int16)
    hi = lax.bitcast_convert_type(t[:, hw:].astype(jnp.bfloat16), jnp.int16)
    lo32 = lo.astype(jnp.int32) & 0xFFFF
    hi32 = hi.astype(jnp.int32) << 16
    return lax.bitcast_convert_type(lo32 | hi32, jnp.float32)


def _transpose_fold_mlp(tbl_t):
    def body(*refs):
        out_ref = refs[-1]
        for f in range(8):
            out_ref[:, f * 16:(f + 1) * 16] = _pack_bf16_halves(
                refs[f][:].T, MLP_DIM)

    in_specs = [
        pl.BlockSpec((MLP_DIM, MLP_B2), functools.partial(
            lambda f, i: (0, f * MLP_NBLK + i), f))
        for f in range(8)
    ]
    return pl.pallas_call(
        body,
        grid=(MLP_NBLK,),
        in_specs=in_specs,
        out_specs=pl.BlockSpec((MLP_B2, 128), lambda i: (i, 0)),
        out_shape=jax.ShapeDtypeStruct((MLP_SPLIT, 128), jnp.float32),
    )(*([tbl_t] * 8))


def _transpose_fold_gmf(tbl_t):
    def body(*refs):
        out_ref = refs[-1]
        for f in range(4):
            out_ref[:, f * 32:(f + 1) * 32] = _pack_bf16_halves(
                refs[f][:].T, GMF_DIM)

    in_specs = [
        pl.BlockSpec((GMF_DIM, GMF_B2), functools.partial(
            lambda f, i: (0, f * GMF_NBLK + i), f))
        for f in range(4)
    ]
    return pl.pallas_call(
        body,
        grid=(GMF_NBLK,),
        in_specs=in_specs,
        out_specs=pl.BlockSpec((GMF_B2, 128), lambda i: (i, 0)),
        out_shape=jax.ShapeDtypeStruct((GMF_SPLIT, 128), jnp.float32),
    )(*([tbl_t] * 4))


def _make_gather_kernel_folded(d, split):
    mesh = plsc.VectorSubcoreMesh(core_axis_name="c", subcore_axis_name="s")
    info = plsc.get_sparse_core_info()
    nc = info.num_cores

    b_per_w = B // NW
    @functools.partial(
        pl.kernel,
        mesh=mesh,
        out_type=[jax.ShapeDtypeStruct((B, d), jnp.float32)],
        scratch_types=[
            pltpu.VMEM((b_per_w,), jnp.int32),
            pltpu.VMEM((CH, 128), jnp.float32),
            pltpu.VMEM((CH, d), jnp.float32),
            pltpu.SemaphoreType.DMA,
        ],
    )
    def gather_kernel(idx_hbm, tbl_hbm, rows_out, idx_v, line_v, row_v, sem):
        wid = lax.axis_index("s") * nc + lax.axis_index("c")
        base = wid * b_per_w
        pltpu.sync_copy(idx_hbm.at[pl.ds(base, b_per_w)], idx_v)

        def chunk(c, _):
            def fire(g, _):
                vec = idx_v[pl.ds(c * CH + g * LN, LN)]
                kvec = lax.rem(vec, split)
                for l in range(LN):
                    pltpu.async_copy(
                        tbl_hbm.at[pl.ds(kvec[l], 1)],
                        line_v.at[pl.ds(g * LN + l, 1)], sem)
                return 0

            lax.fori_loop(0, CH // LN, fire, 0)

            def drain(j, _):
                pltpu.make_async_copy(
                    tbl_hbm.at[pl.ds(0, 1)], line_v.at[pl.ds(j, 1)],
                    sem).wait()
                return 0

            lax.fori_loop(0, CH, drain, 0)

            def extract(g, _):
                vec = idx_v[pl.ds(c * CH + g * LN, LN)]
                ovec = lax.div(vec, split) * d
                for l in range(LN):
                    j = g * LN + l
                    o = ovec[l]
                    for k in range(d // 16):
                        row_v[j, pl.ds(k * 16, 16)] = line_v[
                            j, pl.ds(o + k * 16, 16)]
                return 0

            lax.fori_loop(0, CH // LN, extract, 0)
            pltpu.sync_copy(row_v, rows_out.at[pl.ds(base + c * CH, CH)])
            return 0

        lax.fori_loop(0, b_per_w // CH, chunk, 0)

    return gather_kernel


def _unpack(x):
    bits = lax.bitcast_convert_type(x, jnp.int32)
    lo = lax.bitcast_convert_type(bits << 16, jnp.float32)
    hi = lax.bitcast_convert_type(bits & jnp.int32(-65536), jnp.float32)
    return lo, hi


def _dense_body(gu, gi, mu, mi, w1a_lo, w1a_hi, w1b_lo, w1b_hi, b1, w2, b2,
                w3, b3, wfg_lo, wfg_hi, wfh, bf, out):
    mu_lo, mu_hi = _unpack(mu[:])
    mi_lo, mi_hi = _unpack(mi[:])
    h = jnp.dot(mu_lo, w1a_lo[:], preferred_element_type=jnp.float32)
    h = h + jnp.dot(mu_hi, w1a_hi[:], preferred_element_type=jnp.float32)
    h = h + jnp.dot(mi_lo, w1b_lo[:], preferred_element_type=jnp.float32)
    h = h + jnp.dot(mi_hi, w1b_hi[:], preferred_element_type=jnp.float32)
    h = jnp.maximum(h + b1[:], 0.0)
    h = jnp.maximum(
        jnp.dot(h, w2[:], preferred_element_type=jnp.float32) + b2[:], 0.0)
    h = jnp.maximum(
        jnp.dot(h, w3[:], preferred_element_type=jnp.float32) + b3[:], 0.0)
    gu_lo, gu_hi = _unpack(gu[:])
    gi_lo, gi_hi = _unpack(gi[:])
    s = jnp.dot(gu_lo * gi_lo, wfg_lo[:], preferred_element_type=jnp.float32)
    s = s + jnp.dot(gu_hi * gi_hi, wfg_hi[:],
                    preferred_element_type=jnp.float32)
    s = s + jnp.dot(h, wfh[:], preferred_element_type=jnp.float32)
    out[:] = jax.nn.sigmoid(s + bf[:]).reshape(out.shape)


def kernel(user_indices, item_indices, gmf_user, gmf_item, mlp_user,
           mlp_item, W1, b1, W2, b2, W3, b3, Wf, bf):
    user_indices = user_indices.astype(jnp.int32)
    item_indices = item_indices.astype(jnp.int32)

    gather64 = _make_gather_kernel_folded(GMF_DIM // 2, GMF_SPLIT)
    gather32 = _make_gather_kernel_folded(MLP_DIM // 2, MLP_SPLIT)
    mlp_u2 = _transpose_fold_mlp(mlp_user.T)
    mlp_i2 = _transpose_fold_mlp(mlp_item.T)
    gmf_u2 = _transpose_fold_gmf(gmf_user.T)
    gmf_i2 = _transpose_fold_gmf(gmf_item.T)
    (mu,) = gather32(user_indices, mlp_u2)
    (mi,) = gather32(item_indices, mlp_i2)
    (gu,) = gather64(user_indices, gmf_u2)
    (gi,) = gather64(item_indices, gmf_i2)

    blk = 4096
    grid = B // blk
    hm = MLP_DIM // 2
    hg = GMF_DIM // 2
    w1a_lo = W1[:hm]
    w1a_hi = W1[hm:MLP_DIM]
    w1b_lo = W1[MLP_DIM:MLP_DIM + hm]
    w1b_hi = W1[MLP_DIM + hm:]
    wfg_lo = Wf[:hg]
    wfg_hi = Wf[hg:GMF_DIM]
    wfh = Wf[GMF_DIM:]
    rep = lambda shape: pl.BlockSpec(shape, lambda i: (0, 0))
    out = pl.pallas_call(
        _dense_body,
        grid=(grid,),
        in_specs=[
            pl.BlockSpec((blk, hg), lambda i: (i, 0)),
            pl.BlockSpec((blk, hg), lambda i: (i, 0)),
            pl.BlockSpec((blk, hm), lambda i: (i, 0)),
            pl.BlockSpec((blk, hm), lambda i: (i, 0)),
            rep((hm, 128)),
            rep((hm, 128)),
            rep((hm, 128)),
            rep((hm, 128)),
            rep((1, 128)),
            rep((128, 64)),
            rep((1, 64)),
            rep((64, 32)),
            rep((1, 32)),
            rep((hg, 1)),
            rep((hg, 1)),
            rep((32, 1)),
            rep((1, 1)),
        ],
        out_specs=pl.BlockSpec((blk,), lambda i: (i,)),
        out_shape=jax.ShapeDtypeStruct((B,), jnp.float32),
    )(gu, gi, mu, mi, w1a_lo, w1a_hi, w1b_lo, w1b_hi, b1.reshape(1, -1),
      W2, b2.reshape(1, -1), W3, b3.reshape(1, -1), wfg_lo, wfg_hi, wfh,
      bf.reshape(1, 1))
    return out


# speedup vs baseline: 1.0145x; 1.0145x over previous
import functools

import jax
import jax.numpy as jnp
from jax import lax
from jax.experimental import pallas as pl
from jax.experimental.pallas import tpu as pltpu
from jax.experimental.pallas import tpu_sc as plsc

B = 16384
GMF_DIM = 64
MLP_DIM = 32
CH = 128
LN = 16
NW = 32
MLP_SPLIT = 25088
MLP_B2 = 3584
MLP_NBLK = 7
GMF_SPLIT = 50048
GMF_B2 = 2176
GMF_NBLK = 23


def _transpose_fold_mlp(tbl_t):
    def body(*refs):
        out_ref = refs[-1]
        for f in range(4):
            out_ref[:, f * MLP_DIM:(f + 1) * MLP_DIM] = refs[f][:].T

    in_specs = [
        pl.BlockSpec((MLP_DIM, MLP_B2), functools.partial(
            lambda f, i: (0, f * MLP_NBLK + i), f))
        for f in range(4)
    ]
    return pl.pallas_call(
        body,
        grid=(MLP_NBLK,),
        in_specs=in_specs,
        out_specs=pl.BlockSpec((MLP_B2, 128), lambda i: (i, 0)),
        out_shape=jax.ShapeDtypeStruct((MLP_SPLIT, 128), jnp.float32),
    )(*([tbl_t] * 4))


def _transpose_fold_gmf(tbl_t):
    def body(*refs):
        out_ref = refs[-1]
        for f in range(2):
            out_ref[:, f * GMF_DIM:(f + 1) * GMF_DIM] = refs[f][:].T

    in_specs = [
        pl.BlockSpec((GMF_DIM, GMF_B2), functools.partial(
            lambda f, i: (0, f * GMF_NBLK + i), f))
        for f in range(2)
    ]
    return pl.pallas_call(
        body,
        grid=(GMF_NBLK,),
        in_specs=in_specs,
        out_specs=pl.BlockSpec((GMF_B2, 128), lambda i: (i, 0)),
        out_shape=jax.ShapeDtypeStruct((GMF_SPLIT, 128), jnp.float32),
    )(*([tbl_t] * 2))


def _make_gather_kernel_folded(d, split):
    mesh = plsc.VectorSubcoreMesh(core_axis_name="c", subcore_axis_name="s")
    info = plsc.get_sparse_core_info()
    nc = info.num_cores

    b_per_w = B // NW
    @functools.partial(
        pl.kernel,
        mesh=mesh,
        out_type=[jax.ShapeDtypeStruct((B, d), jnp.float32)],
        scratch_types=[
            pltpu.VMEM((b_per_w,), jnp.int32),
            pltpu.VMEM((CH, 128), jnp.float32),
            pltpu.VMEM((CH, d), jnp.float32),
            pltpu.SemaphoreType.DMA,
        ],
    )
    def gather_kernel(idx_hbm, tbl_hbm, rows_out, idx_v, line_v, row_v, sem):
        wid = lax.axis_index("s") * nc + lax.axis_index("c")
        base = wid * b_per_w
        pltpu.sync_copy(idx_hbm.at[pl.ds(base, b_per_w)], idx_v)

        def chunk(c, _):
            def fire(g, _):
                vec = idx_v[pl.ds(c * CH + g * LN, LN)]
                kvec = lax.rem(vec, split)
                for l in range(LN):
                    pltpu.async_copy(
                        tbl_hbm.at[pl.ds(kvec[l], 1)],
                        line_v.at[pl.ds(g * LN + l, 1)], sem)
                return 0

            lax.fori_loop(0, CH // LN, fire, 0)

            def drain(j, _):
                pltpu.make_async_copy(
                    tbl_hbm.at[pl.ds(0, 1)], line_v.at[pl.ds(j, 1)],
                    sem).wait()
                return 0

            lax.fori_loop(0, CH, drain, 0)

            def extract(g, _):
                vec = idx_v[pl.ds(c * CH + g * LN, LN)]
                ovec = lax.div(vec, split) * d
                for l in range(LN):
                    j = g * LN + l
                    o = ovec[l]
                    for k in range(d // 16):
                        row_v[j, pl.ds(k * 16, 16)] = line_v[
                            j, pl.ds(o + k * 16, 16)]
                return 0

            lax.fori_loop(0, CH // LN, extract, 0)
            pltpu.sync_copy(row_v, rows_out.at[pl.ds(base + c * CH, CH)])
            return 0

        lax.fori_loop(0, b_per_w // CH, chunk, 0)

    return gather_kernel


def _dense_body(gu, gi, mu, mi, w1a, w1b, b1, w2, b2, w3, b3, wfg, wfh, bf,
                out):
    h = jnp.dot(mu[:], w1a[:], preferred_element_type=jnp.float32)
    h = h + jnp.dot(mi[:], w1b[:], preferred_element_type=jnp.float32)
    h = jnp.maximum(h + b1[:], 0.0)
    h = jnp.maximum(
        jnp.dot(h, w2[:], preferred_element_type=jnp.float32) + b2[:], 0.0)
    h = jnp.maximum(
        jnp.dot(h, w3[:], preferred_element_type=jnp.float32) + b3[:], 0.0)
    g = gu[:] * gi[:]
    s = jnp.dot(g, wfg[:], preferred_element_type=jnp.float32)
    s = s + jnp.dot(h, wfh[:], preferred_element_type=jnp.float32)
    out[:] = jax.nn.sigmoid(s + bf[:]).reshape(out.shape)


def kernel(user_indices, item_indices, gmf_user, gmf_item, mlp_user,
           mlp_item, W1, b1, W2, b2, W3, b3, Wf, bf):
    user_indices = user_indices.astype(jnp.int32)
    item_indices = item_indices.astype(jnp.int32)

    gather64 = _make_gather_kernel_folded(GMF_DIM, GMF_SPLIT)
    gather32 = _make_gather_kernel_folded(MLP_DIM, MLP_SPLIT)
    mlp_u2 = _transpose_fold_mlp(mlp_user.T)
    mlp_i2 = _transpose_fold_mlp(mlp_item.T)
    gmf_u2 = _transpose_fold_gmf(gmf_user.T)
    gmf_i2 = _transpose_fold_gmf(gmf_item.T)
    (mu,) = gather32(user_indices, mlp_u2)
    (mi,) = gather32(item_indices, mlp_i2)
    (gu,) = gather64(user_indices, gmf_u2)
    (gi,) = gather64(item_indices, gmf_i2)

    blk = 4096
    grid = B // blk
    w1a = W1[:MLP_DIM]
    w1b = W1[MLP_DIM:]
    wfg = Wf[:GMF_DIM]
    wfh = Wf[GMF_DIM:]
    rep = lambda shape: pl.BlockSpec(shape, lambda i: (0, 0))
    out = pl.pallas_call(
        _dense_body,
        grid=(grid,),
        in_specs=[
            pl.BlockSpec((blk, GMF_DIM), lambda i: (i, 0)),
            pl.BlockSpec((blk, GMF_DIM), lambda i: (i, 0)),
            pl.BlockSpec((blk, MLP_DIM), lambda i: (i, 0)),
            pl.BlockSpec((blk, MLP_DIM), lambda i: (i, 0)),
            rep((MLP_DIM, 128)),
            rep((MLP_DIM, 128)),
            rep((1, 128)),
            rep((128, 64)),
            rep((1, 64)),
            rep((64, 32)),
            rep((1, 32)),
            rep((GMF_DIM, 1)),
            rep((32, 1)),
            rep((1, 1)),
        ],
        out_specs=pl.BlockSpec((blk,), lambda i: (i,)),
        out_shape=jax.ShapeDtypeStruct((B,), jnp.float32),
    )(gu, gi, mu, mi, w1a, w1b, b1.reshape(1, -1), W2, b2.reshape(1, -1),
      W3, b3.reshape(1, -1), wfg, wfh, bf.reshape(1, 1))
    return out
